# Initial kernel scaffold; baseline (speedup 1.0000x reference)
import jax
import jax.numpy as jnp
from jax.experimental import pallas as pl


def kernel(node_feats, edge_index, W, b):
    raise NotImplementedError("write your pallas kernel here")


# speedup vs baseline: 8.5432x; 8.5432x over previous
import functools

import jax
import jax.numpy as jnp
from jax import lax
from jax.experimental import pallas as pl
from jax.experimental.pallas import tpu as pltpu
from jax.experimental.pallas import tpu_sc as plsc

N = 10000
E = 160000
EP = 16 * 80 * 128
NPAD = 10240

def _deg_body(dstk, zeros640, out, dvm, ovm, vmf, vmc, dsh):
    c = lax.axis_index("c")
    s = lax.axis_index("s")
    wid = c * 16 + s
    ones16 = jnp.ones((16,), jnp.float32)

    @pl.loop(0, 8)
    def _(k):
        ovm[pl.ds(k * 16, 16)] = ones16

    pltpu.sync_copy(zeros640, dsh.at[pl.ds(s * 640, 640)])
    pltpu.sync_copy(dstk.at[wid], dvm)
    plsc.subcore_barrier()

    @pl.loop(0, 40)
    def _(j):
        pltpu.sync_copy(ovm, dsh.at[dvm.at[j]], add=True)

    plsc.subcore_barrier()
    pltpu.sync_copy(dsh.at[pl.ds(s * 640, 640)], vmf)

    @pl.loop(0, 40)
    def _(i):
        vmc[i // 8, pl.ds((i % 8) * 16, 16)] = vmf[pl.ds(i * 16, 16)]

    pltpu.sync_copy(vmc, out.at[wid])


def _k2_body(x_ref, w_ref, b_ref, hist_ref, whs_ref, norm_ref):
    c = pl.program_id(1)
    degs = jnp.sum(hist_ref[...], axis=1)
    norm = lax.rsqrt(jnp.maximum(degs, 1.0))
    acc = lax.dot_general(
        x_ref[...], w_ref[...], (((1,), (1,)), ((), ())),
        preferred_element_type=jnp.float32)
    whs_ref[0] = (acc + b_ref[pl.ds(c, 1), :]) * norm[:, None]
    norm_ref[...] = norm[:, None]


def _k2(x, w, b2, hist):
    return pl.pallas_call(
        _k2_body,
        grid=(5, 2),
        in_specs=[
            pl.BlockSpec((2000, 256), lambda i, c: (i, 0)),
            pl.BlockSpec((128, 256), lambda i, c: (c, 0)),
            pl.BlockSpec((2, 128), lambda i, c: (0, 0)),
            pl.BlockSpec((2000, 2), lambda i, c: (i, 0)),
        ],
        out_specs=[
            pl.BlockSpec((1, 2000, 128), lambda i, c: (c, i, 0)),
            pl.BlockSpec((2000, 1), lambda i, c: (i, 0)),
        ],
        out_shape=[
            jax.ShapeDtypeStruct((2, N, 128), jnp.float32),
            jax.ShapeDtypeStruct((N, 1), jnp.float32),
        ],
    )(x, w, b2, hist)


def _msg_body(whs_flat, src4, dst3, zeros128, out, svm, dvm, rows, hsh, sem):
    c = lax.axis_index("c")
    s = lax.axis_index("s")
    wid = c * 16 + s
    for k in range(5):
        pltpu.sync_copy(zeros128, hsh.at[pl.ds(s * 640 + k * 128, 128)])
    pltpu.sync_copy(src4.at[wid], svm)
    pltpu.sync_copy(dst3.at[s], dvm)
    plsc.subcore_barrier()

    @pl.loop(0, 80)
    def _(j):
        pltpu.async_copy(whs_flat.at[svm.at[j]], rows, sem).wait()
        pltpu.sync_copy(rows, hsh.at[dvm.at[j]], add=True)

    plsc.subcore_barrier()
    pltpu.sync_copy(hsh.at[pl.ds(s * 625, 625)], out.at[wid])


def _k4_body(h2_ref, norm_ref, x_ref, o_ref):
    h = jnp.concatenate([h2_ref[0], h2_ref[1]], axis=1)
    o_ref[...] = jnp.maximum(h * norm_ref[...], 0.0) + x_ref[...]


@functools.lru_cache(maxsize=None)
def _sc_kernels():
    mesh = plsc.VectorSubcoreMesh(
        core_axis_name="c", subcore_axis_name="s", num_cores=2,
        num_subcores=16)
    deg = pl.kernel(
        _deg_body,
        mesh=mesh,
        out_type=jax.ShapeDtypeStruct((32, 5, 128), jnp.float32),
        scratch_types=[
            pltpu.VMEM((40, 128), jnp.int32),
            pltpu.VMEM((128,), jnp.float32),
            pltpu.VMEM((640,), jnp.float32),
            pltpu.VMEM((5, 128), jnp.float32),
            pltpu.VMEM_SHARED((NPAD,), jnp.float32),
        ],
    )
    msg = pl.kernel(
        _msg_body,
        mesh=mesh,
        out_type=jax.ShapeDtypeStruct((32, 625, 128), jnp.float32),
        scratch_types=[
            pltpu.VMEM((80, 128), jnp.int32),
            pltpu.VMEM((80, 128), jnp.int32),
            pltpu.VMEM((128, 128), jnp.float32),
            pltpu.VMEM_SHARED((NPAD, 128), jnp.float32),
            pltpu.SemaphoreType.DMA,
        ],
    )
    return deg, msg


def _k4(h2, norm, x):
    return pl.pallas_call(
        _k4_body,
        grid=(5,),
        in_specs=[
            pl.BlockSpec((2, 2000, 128), lambda i: (0, i, 0)),
            pl.BlockSpec((2000, 1), lambda i: (i, 0)),
            pl.BlockSpec((2000, 256), lambda i: (i, 0)),
        ],
        out_specs=pl.BlockSpec((2000, 256), lambda i: (i, 0)),
        out_shape=jax.ShapeDtypeStruct((N, 256), jnp.float32),
    )(h2, norm, x)


def kernel(node_feats, edge_index, W, b):
    src = edge_index[0].astype(jnp.int32)
    dst = edge_index[1].astype(jnp.int32)
    pad = EP - E
    ar = jnp.arange(pad, dtype=jnp.int32)
    srcp = jnp.concatenate([src, (ar * 37) % N])
    dstp = jnp.concatenate([dst, N + (ar % 240)])
    dstk1 = dstp.reshape(32, 40, 128)
    dst3 = dstp.reshape(16, 80, 128)
    src4 = jnp.stack([srcp, srcp + N]).reshape(32, 80, 128)
    zeros640 = jnp.zeros((640,), jnp.float32)
    zeros128 = jnp.zeros((128, 128), jnp.float32)

    deg_kernel, msg_kernel = _sc_kernels()
    hist = deg_kernel(dstk1, zeros640).reshape(2, NPAD)[:, :N].T
    whs, norm = _k2(node_feats, W, b.reshape(2, 128), hist)
    h32 = msg_kernel(whs.reshape(2 * N, 128), src4, dst3, zeros128)
    return _k4(h32.reshape(2, N, 128), norm, node_feats)


# speedup vs baseline: 11.3465x; 1.3281x over previous
import functools

import jax
import jax.numpy as jnp
from jax import lax
from jax.experimental import pallas as pl
from jax.experimental.pallas import tpu as pltpu
from jax.experimental.pallas import tpu_sc as plsc

N = 10000
E = 160000
EP = 16 * 80 * 128
NPAD = 10240

def _deg_body(dstk, zeros640, out, dvm, ovm, vmf, vmc, dsh):
    c = lax.axis_index("c")
    s = lax.axis_index("s")
    wid = c * 16 + s
    ones16 = jnp.ones((16,), jnp.float32)

    @pl.loop(0, 8)
    def _(k):
        ovm[pl.ds(k * 16, 16)] = ones16

    pltpu.sync_copy(zeros640, dsh.at[pl.ds(s * 640, 640)])
    pltpu.sync_copy(dstk.at[wid], dvm)
    plsc.subcore_barrier()

    @pl.loop(0, 40)
    def _(j):
        pltpu.sync_copy(ovm, dsh.at[dvm.at[j]], add=True)

    plsc.subcore_barrier()
    pltpu.sync_copy(dsh.at[pl.ds(s * 640, 640)], vmf)

    @pl.loop(0, 40)
    def _(i):
        vmc[i // 8, pl.ds((i % 8) * 16, 16)] = vmf[pl.ds(i * 16, 16)]

    pltpu.sync_copy(vmc, out.at[wid])


def _k2_body(x_ref, w_ref, b_ref, hist_ref, whs_ref, norm_ref):
    c = pl.program_id(1)
    degs = jnp.sum(hist_ref[...], axis=1)
    norm = lax.rsqrt(jnp.maximum(degs, 1.0))
    acc = lax.dot_general(
        x_ref[...], w_ref[...], (((1,), (1,)), ((), ())),
        preferred_element_type=jnp.float32)
    whs_ref[0] = (acc + b_ref[pl.ds(c, 1), :]) * norm[:, None]
    norm_ref[...] = norm[:, None]


def _k2(x, w, b2, hist):
    return pl.pallas_call(
        _k2_body,
        grid=(5, 2),
        in_specs=[
            pl.BlockSpec((2000, 256), lambda i, c: (i, 0)),
            pl.BlockSpec((128, 256), lambda i, c: (c, 0)),
            pl.BlockSpec((2, 128), lambda i, c: (0, 0)),
            pl.BlockSpec((2000, 2), lambda i, c: (i, 0)),
        ],
        out_specs=[
            pl.BlockSpec((1, 2000, 128), lambda i, c: (c, i, 0)),
            pl.BlockSpec((2000, 1), lambda i, c: (i, 0)),
        ],
        out_shape=[
            jax.ShapeDtypeStruct((2, N, 128), jnp.float32),
            jax.ShapeDtypeStruct((N, 1), jnp.float32),
        ],
    )(x, w, b2, hist)


def _msg_body(whs_flat, src4, dst3, zeros128, out, svm, dvm, rows0, rows1,
              hsh, sg0, sg1, ss0, ss1):
    c = lax.axis_index("c")
    s = lax.axis_index("s")
    wid = c * 16 + s
    for k in range(5):
        pltpu.sync_copy(zeros128, hsh.at[pl.ds(s * 640 + k * 128, 128)])
    plsc.subcore_barrier()

    for p in range(2):
        pltpu.sync_copy(src4.at[wid, pl.ds(p * 40, 40)], svm)
        pltpu.sync_copy(dst3.at[s, pl.ds(p * 40, 40)], dvm)
        pltpu.async_copy(whs_flat.at[svm.at[0]], rows0, sg0)

        @pl.loop(0, 20)
        def _(j2):
            a = 2 * j2

            @pl.when(j2 > 0)
            def _():
                pltpu.make_async_copy(rows1, hsh.at[dvm.at[0]], ss1).wait()

            pltpu.async_copy(whs_flat.at[svm.at[a + 1]], rows1, sg1)
            pltpu.make_async_copy(whs_flat.at[svm.at[0]], rows0, sg0).wait()
            pltpu.async_copy(rows0, hsh.at[dvm.at[a]], ss0, add=True)

            @pl.when(j2 < 19)
            def _():
                pltpu.make_async_copy(rows0, hsh.at[dvm.at[0]], ss0).wait()
                pltpu.async_copy(whs_flat.at[svm.at[a + 2]], rows0, sg0)

            pltpu.make_async_copy(whs_flat.at[svm.at[0]], rows1, sg1).wait()
            pltpu.async_copy(rows1, hsh.at[dvm.at[a + 1]], ss1, add=True)

        pltpu.make_async_copy(rows0, hsh.at[dvm.at[0]], ss0).wait()
        pltpu.make_async_copy(rows1, hsh.at[dvm.at[0]], ss1).wait()

    plsc.subcore_barrier()
    pltpu.sync_copy(hsh.at[pl.ds(s * 625, 625)], out.at[wid])


def _k4_body(h2_ref, norm_ref, x_ref, o_ref):
    h = jnp.concatenate([h2_ref[0], h2_ref[1]], axis=1)
    o_ref[...] = jnp.maximum(h * norm_ref[...], 0.0) + x_ref[...]


@functools.lru_cache(maxsize=None)
def _sc_kernels():
    mesh = plsc.VectorSubcoreMesh(
        core_axis_name="c", subcore_axis_name="s", num_cores=2,
        num_subcores=16)
    deg = pl.kernel(
        _deg_body,
        mesh=mesh,
        out_type=jax.ShapeDtypeStruct((32, 5, 128), jnp.float32),
        scratch_types=[
            pltpu.VMEM((40, 128), jnp.int32),
            pltpu.VMEM((128,), jnp.float32),
            pltpu.VMEM((640,), jnp.float32),
            pltpu.VMEM((5, 128), jnp.float32),
            pltpu.VMEM_SHARED((NPAD,), jnp.float32),
        ],
    )
    msg = pl.kernel(
        _msg_body,
        mesh=mesh,
        out_type=jax.ShapeDtypeStruct((32, 625, 128), jnp.float32),
        scratch_types=[
            pltpu.VMEM((40, 128), jnp.int32),
            pltpu.VMEM((40, 128), jnp.int32),
            pltpu.VMEM((128, 128), jnp.float32),
            pltpu.VMEM((128, 128), jnp.float32),
            pltpu.VMEM_SHARED((NPAD, 128), jnp.float32),
            pltpu.SemaphoreType.DMA,
            pltpu.SemaphoreType.DMA,
            pltpu.SemaphoreType.DMA,
            pltpu.SemaphoreType.DMA,
        ],
    )
    return deg, msg


def _k4(h2, norm, x):
    return pl.pallas_call(
        _k4_body,
        grid=(5,),
        in_specs=[
            pl.BlockSpec((2, 2000, 128), lambda i: (0, i, 0)),
            pl.BlockSpec((2000, 1), lambda i: (i, 0)),
            pl.BlockSpec((2000, 256), lambda i: (i, 0)),
        ],
        out_specs=pl.BlockSpec((2000, 256), lambda i: (i, 0)),
        out_shape=jax.ShapeDtypeStruct((N, 256), jnp.float32),
    )(h2, norm, x)


def kernel(node_feats, edge_index, W, b):
    src = edge_index[0].astype(jnp.int32)
    dst = edge_index[1].astype(jnp.int32)
    pad = EP - E
    ar = jnp.arange(pad, dtype=jnp.int32)
    srcp = jnp.concatenate([src, (ar * 37) % N])
    dstp = jnp.concatenate([dst, N + (ar % 240)])
    dstk1 = dstp.reshape(32, 40, 128)
    dst3 = dstp.reshape(16, 80, 128)
    src4 = jnp.stack([srcp, srcp + N]).reshape(32, 80, 128)
    zeros640 = jnp.zeros((640,), jnp.float32)
    zeros128 = jnp.zeros((128, 128), jnp.float32)

    deg_kernel, msg_kernel = _sc_kernels()
    hist = deg_kernel(dstk1, zeros640).reshape(2, NPAD)[:, :N].T
    whs, norm = _k2(node_feats, W, b.reshape(2, 128), hist)
    h32 = msg_kernel(whs.reshape(2 * N, 128), src4, dst3, zeros128)
    return _k4(h32.reshape(2, N, 128), norm, node_feats)


# speedup vs baseline: 11.3677x; 1.0019x over previous
import functools

import jax
import jax.numpy as jnp
from jax import lax
from jax.experimental import pallas as pl
from jax.experimental.pallas import tpu as pltpu
from jax.experimental.pallas import tpu_sc as plsc

N = 10000
E = 160000
EP = 16 * 80 * 128
NPAD = 10240

def _deg_body(dstk, zeros640, out, dvm, ovm, vmf, vmc, dsh):
    c = lax.axis_index("c")
    s = lax.axis_index("s")
    wid = c * 16 + s
    ones16 = jnp.ones((16,), jnp.float32)

    @pl.loop(0, 8)
    def _(k):
        ovm[pl.ds(k * 16, 16)] = ones16

    pltpu.sync_copy(zeros640, dsh.at[pl.ds(s * 640, 640)])
    pltpu.sync_copy(dstk.at[wid], dvm)
    plsc.subcore_barrier()

    @pl.loop(0, 40)
    def _(j):
        pltpu.sync_copy(ovm, dsh.at[dvm.at[j]], add=True)

    plsc.subcore_barrier()
    pltpu.sync_copy(dsh.at[pl.ds(s * 640, 640)], vmf)

    @pl.loop(0, 40)
    def _(i):
        vmc[i // 8, pl.ds((i % 8) * 16, 16)] = vmf[pl.ds(i * 16, 16)]

    pltpu.sync_copy(vmc, out.at[wid])


def _k2_body(x_ref, w_ref, b_ref, hist_ref, whs_ref, norm_ref):
    c = pl.program_id(1)
    degs = jnp.sum(hist_ref[...], axis=1)
    norm = lax.rsqrt(jnp.maximum(degs, 1.0))
    acc = lax.dot_general(
        x_ref[...], w_ref[...], (((1,), (1,)), ((), ())),
        preferred_element_type=jnp.float32)
    whs_ref[0] = (acc + b_ref[pl.ds(c, 1), :]) * norm[:, None]
    norm_ref[...] = norm[:, None]


def _k2(x, w, b2, hist):
    return pl.pallas_call(
        _k2_body,
        grid=(5, 2),
        in_specs=[
            pl.BlockSpec((2000, 256), lambda i, c: (i, 0)),
            pl.BlockSpec((128, 256), lambda i, c: (c, 0)),
            pl.BlockSpec((2, 128), lambda i, c: (0, 0)),
            pl.BlockSpec((2000, 2), lambda i, c: (i, 0)),
        ],
        out_specs=[
            pl.BlockSpec((1, 2000, 128), lambda i, c: (c, i, 0)),
            pl.BlockSpec((2000, 1), lambda i, c: (i, 0)),
        ],
        out_shape=[
            jax.ShapeDtypeStruct((2, N, 128), jnp.float32),
            jax.ShapeDtypeStruct((N, 1), jnp.float32),
        ],
    )(x, w, b2, hist)


def _msg_body(whs_flat, src4, dst3, zeros128, out, svm, dvm, rows0, rows1,
              hsh, sg0, sg1, ss0, ss1):
    c = lax.axis_index("c")
    s = lax.axis_index("s")
    wid = c * 16 + s
    for k in range(5):
        pltpu.sync_copy(zeros128, hsh.at[pl.ds(s * 640 + k * 128, 128)])
    plsc.subcore_barrier()

    def gstart(j, buf, sem):
        pltpu.async_copy(
            whs_flat.at[svm.at[j, pl.ds(0, 64)]], buf.at[pl.ds(0, 64)], sem)
        pltpu.async_copy(
            whs_flat.at[svm.at[j, pl.ds(64, 64)]], buf.at[pl.ds(64, 64)], sem)

    def gwait(buf, sem):
        pltpu.make_async_copy(
            whs_flat.at[svm.at[0, pl.ds(0, 64)]], buf.at[pl.ds(0, 64)],
            sem).wait()
        pltpu.make_async_copy(
            whs_flat.at[svm.at[0, pl.ds(64, 64)]], buf.at[pl.ds(64, 64)],
            sem).wait()

    for p in range(2):
        pltpu.sync_copy(src4.at[wid, pl.ds(p * 40, 40)], svm)
        pltpu.sync_copy(dst3.at[s, pl.ds(p * 40, 40)], dvm)
        gstart(0, rows0, sg0)

        @pl.loop(0, 20)
        def _(j2):
            a = 2 * j2

            @pl.when(j2 > 0)
            def _():
                pltpu.make_async_copy(rows1, hsh.at[dvm.at[0]], ss1).wait()

            gstart(a + 1, rows1, sg1)
            gwait(rows0, sg0)
            pltpu.async_copy(rows0, hsh.at[dvm.at[a]], ss0, add=True)

            @pl.when(j2 < 19)
            def _():
                pltpu.make_async_copy(rows0, hsh.at[dvm.at[0]], ss0).wait()
                gstart(a + 2, rows0, sg0)

            gwait(rows1, sg1)
            pltpu.async_copy(rows1, hsh.at[dvm.at[a + 1]], ss1, add=True)

        pltpu.make_async_copy(rows0, hsh.at[dvm.at[0]], ss0).wait()
        pltpu.make_async_copy(rows1, hsh.at[dvm.at[0]], ss1).wait()

    plsc.subcore_barrier()
    pltpu.sync_copy(hsh.at[pl.ds(s * 625, 625)], out.at[wid])


def _k4_body(h2_ref, norm_ref, x_ref, o_ref):
    h = jnp.concatenate([h2_ref[0], h2_ref[1]], axis=1)
    o_ref[...] = jnp.maximum(h * norm_ref[...], 0.0) + x_ref[...]


@functools.lru_cache(maxsize=None)
def _sc_kernels():
    mesh = plsc.VectorSubcoreMesh(
        core_axis_name="c", subcore_axis_name="s", num_cores=2,
        num_subcores=16)
    deg = pl.kernel(
        _deg_body,
        mesh=mesh,
        out_type=jax.ShapeDtypeStruct((32, 5, 128), jnp.float32),
        scratch_types=[
            pltpu.VMEM((40, 128), jnp.int32),
            pltpu.VMEM((128,), jnp.float32),
            pltpu.VMEM((640,), jnp.float32),
            pltpu.VMEM((5, 128), jnp.float32),
            pltpu.VMEM_SHARED((NPAD,), jnp.float32),
        ],
    )
    msg = pl.kernel(
        _msg_body,
        mesh=mesh,
        out_type=jax.ShapeDtypeStruct((32, 625, 128), jnp.float32),
        scratch_types=[
            pltpu.VMEM((40, 128), jnp.int32),
            pltpu.VMEM((40, 128), jnp.int32),
            pltpu.VMEM((128, 128), jnp.float32),
            pltpu.VMEM((128, 128), jnp.float32),
            pltpu.VMEM_SHARED((NPAD, 128), jnp.float32),
            pltpu.SemaphoreType.DMA,
            pltpu.SemaphoreType.DMA,
            pltpu.SemaphoreType.DMA,
            pltpu.SemaphoreType.DMA,
        ],
    )
    return deg, msg


def _k4(h2, norm, x):
    return pl.pallas_call(
        _k4_body,
        grid=(5,),
        in_specs=[
            pl.BlockSpec((2, 2000, 128), lambda i: (0, i, 0)),
            pl.BlockSpec((2000, 1), lambda i: (i, 0)),
            pl.BlockSpec((2000, 256), lambda i: (i, 0)),
        ],
        out_specs=pl.BlockSpec((2000, 256), lambda i: (i, 0)),
        out_shape=jax.ShapeDtypeStruct((N, 256), jnp.float32),
    )(h2, norm, x)


def kernel(node_feats, edge_index, W, b):
    src = edge_index[0].astype(jnp.int32)
    dst = edge_index[1].astype(jnp.int32)
    pad = EP - E
    ar = jnp.arange(pad, dtype=jnp.int32)
    srcp = jnp.concatenate([src, (ar * 37) % N])
    dstp = jnp.concatenate([dst, N + (ar % 240)])
    dstk1 = dstp.reshape(32, 40, 128)
    dst3 = dstp.reshape(16, 80, 128)
    src4 = jnp.stack([srcp, srcp + N]).reshape(32, 80, 128)
    zeros640 = jnp.zeros((640,), jnp.float32)
    zeros128 = jnp.zeros((128, 128), jnp.float32)

    deg_kernel, msg_kernel = _sc_kernels()
    hist = deg_kernel(dstk1, zeros640).reshape(2, NPAD)[:, :N].T
    whs, norm = _k2(node_feats, W, b.reshape(2, 128), hist)
    h32 = msg_kernel(whs.reshape(2 * N, 128), src4, dst3, zeros128)
    return _k4(h32.reshape(2, N, 128), norm, node_feats)


# speedup vs baseline: 11.3786x; 1.0010x over previous
import functools

import jax
import jax.numpy as jnp
from jax import lax
from jax.experimental import pallas as pl
from jax.experimental.pallas import tpu as pltpu
from jax.experimental.pallas import tpu_sc as plsc

N = 10000
E = 160000
EP = 16 * 80 * 128
NPAD = 10240

def _deg_body(dstk, zeros640, out, dvm, ovm, vmf, vmc, dsh):
    c = lax.axis_index("c")
    s = lax.axis_index("s")
    wid = c * 16 + s
    ones16 = jnp.ones((16,), jnp.float32)

    @pl.loop(0, 8)
    def _(k):
        ovm[pl.ds(k * 16, 16)] = ones16

    pltpu.sync_copy(zeros640, dsh.at[pl.ds(s * 640, 640)])
    pltpu.sync_copy(dstk.at[wid], dvm)
    plsc.subcore_barrier()

    @pl.loop(0, 40)
    def _(j):
        pltpu.sync_copy(ovm, dsh.at[dvm.at[j]], add=True)

    plsc.subcore_barrier()
    pltpu.sync_copy(dsh.at[pl.ds(s * 640, 640)], vmf)

    @pl.loop(0, 40)
    def _(i):
        vmc[i // 8, pl.ds((i % 8) * 16, 16)] = vmf[pl.ds(i * 16, 16)]

    pltpu.sync_copy(vmc, out.at[wid])


def _k2_body(x_ref, w_ref, b_ref, hist_ref, whs_ref, norm_ref):
    c = pl.program_id(1)
    degs = jnp.sum(hist_ref[...], axis=1)
    norm = lax.rsqrt(jnp.maximum(degs, 1.0))
    acc = lax.dot_general(
        x_ref[...], w_ref[...], (((1,), (1,)), ((), ())),
        preferred_element_type=jnp.float32)
    whs_ref[0] = (acc + b_ref[pl.ds(c, 1), :]) * norm[:, None]
    norm_ref[...] = norm[:, None]


def _k2(x, w, b2, hist):
    return pl.pallas_call(
        _k2_body,
        grid=(5, 2),
        in_specs=[
            pl.BlockSpec((2000, 256), lambda i, c: (i, 0)),
            pl.BlockSpec((128, 256), lambda i, c: (c, 0)),
            pl.BlockSpec((2, 128), lambda i, c: (0, 0)),
            pl.BlockSpec((2000, 2), lambda i, c: (i, 0)),
        ],
        out_specs=[
            pl.BlockSpec((1, 2000, 128), lambda i, c: (c, i, 0)),
            pl.BlockSpec((2000, 1), lambda i, c: (i, 0)),
        ],
        out_shape=[
            jax.ShapeDtypeStruct((2, N, 128), jnp.float32),
            jax.ShapeDtypeStruct((N, 1), jnp.float32),
        ],
    )(x, w, b2, hist)


def _msg_body(whs_flat, src4, dst3, zeros128, out, svm, dvm, rows0, rows1,
              hsh, sg0, sg1, ss0, ss1):
    c = lax.axis_index("c")
    s = lax.axis_index("s")
    wid = c * 16 + s
    for k in range(5):
        pltpu.sync_copy(zeros128, hsh.at[pl.ds(s * 640 + k * 128, 128)])
    plsc.subcore_barrier()

    def gstart(j, buf, sem):
        pltpu.async_copy(
            whs_flat.at[svm.at[j, pl.ds(0, 64)]], buf.at[pl.ds(0, 64)], sem)
        pltpu.async_copy(
            whs_flat.at[svm.at[j, pl.ds(64, 64)]], buf.at[pl.ds(64, 64)], sem)

    def gwait(buf, sem):
        pltpu.make_async_copy(
            whs_flat.at[svm.at[0, pl.ds(0, 64)]], buf.at[pl.ds(0, 64)],
            sem).wait()
        pltpu.make_async_copy(
            whs_flat.at[svm.at[0, pl.ds(64, 64)]], buf.at[pl.ds(64, 64)],
            sem).wait()

    for p in range(2):
        pltpu.sync_copy(src4.at[wid, pl.ds(p * 40, 40)], svm)
        pltpu.sync_copy(dst3.at[s, pl.ds(p * 40, 40)], dvm)
        gstart(0, rows0, sg0)

        @pl.loop(0, 20)
        def _(j2):
            a = 2 * j2

            @pl.when(j2 > 0)
            def _():
                pltpu.make_async_copy(rows1, hsh.at[dvm.at[0]], ss1).wait()

            gstart(a + 1, rows1, sg1)
            gwait(rows0, sg0)
            pltpu.async_copy(rows0.at[pl.ds(0, 64)],
                             hsh.at[dvm.at[a, pl.ds(0, 64)]], ss0, add=True)
            pltpu.async_copy(rows0.at[pl.ds(64, 64)],
                             hsh.at[dvm.at[a, pl.ds(64, 64)]], ss0, add=True)

            @pl.when(j2 < 19)
            def _():
                pltpu.make_async_copy(rows0, hsh.at[dvm.at[0]], ss0).wait()
                gstart(a + 2, rows0, sg0)

            gwait(rows1, sg1)
            pltpu.async_copy(rows1.at[pl.ds(0, 64)],
                             hsh.at[dvm.at[a + 1, pl.ds(0, 64)]], ss1,
                             add=True)
            pltpu.async_copy(rows1.at[pl.ds(64, 64)],
                             hsh.at[dvm.at[a + 1, pl.ds(64, 64)]], ss1,
                             add=True)

        pltpu.make_async_copy(rows0, hsh.at[dvm.at[0]], ss0).wait()
        pltpu.make_async_copy(rows1, hsh.at[dvm.at[0]], ss1).wait()

    plsc.subcore_barrier()
    pltpu.sync_copy(hsh.at[pl.ds(s * 625, 625)], out.at[wid])


def _k4_body(h2_ref, norm_ref, x_ref, o_ref):
    h = jnp.concatenate([h2_ref[0], h2_ref[1]], axis=1)
    o_ref[...] = jnp.maximum(h * norm_ref[...], 0.0) + x_ref[...]


@functools.lru_cache(maxsize=None)
def _sc_kernels():
    mesh = plsc.VectorSubcoreMesh(
        core_axis_name="c", subcore_axis_name="s", num_cores=2,
        num_subcores=16)
    deg = pl.kernel(
        _deg_body,
        mesh=mesh,
        out_type=jax.ShapeDtypeStruct((32, 5, 128), jnp.float32),
        scratch_types=[
            pltpu.VMEM((40, 128), jnp.int32),
            pltpu.VMEM((128,), jnp.float32),
            pltpu.VMEM((640,), jnp.float32),
            pltpu.VMEM((5, 128), jnp.float32),
            pltpu.VMEM_SHARED((NPAD,), jnp.float32),
        ],
    )
    msg = pl.kernel(
        _msg_body,
        mesh=mesh,
        out_type=jax.ShapeDtypeStruct((32, 625, 128), jnp.float32),
        scratch_types=[
            pltpu.VMEM((40, 128), jnp.int32),
            pltpu.VMEM((40, 128), jnp.int32),
            pltpu.VMEM((128, 128), jnp.float32),
            pltpu.VMEM((128, 128), jnp.float32),
            pltpu.VMEM_SHARED((NPAD, 128), jnp.float32),
            pltpu.SemaphoreType.DMA,
            pltpu.SemaphoreType.DMA,
            pltpu.SemaphoreType.DMA,
            pltpu.SemaphoreType.DMA,
        ],
    )
    return deg, msg


def _k4(h2, norm, x):
    return pl.pallas_call(
        _k4_body,
        grid=(5,),
        in_specs=[
            pl.BlockSpec((2, 2000, 128), lambda i: (0, i, 0)),
            pl.BlockSpec((2000, 1), lambda i: (i, 0)),
            pl.BlockSpec((2000, 256), lambda i: (i, 0)),
        ],
        out_specs=pl.BlockSpec((2000, 256), lambda i: (i, 0)),
        out_shape=jax.ShapeDtypeStruct((N, 256), jnp.float32),
    )(h2, norm, x)


def kernel(node_feats, edge_index, W, b):
    src = edge_index[0].astype(jnp.int32)
    dst = edge_index[1].astype(jnp.int32)
    pad = EP - E
    ar = jnp.arange(pad, dtype=jnp.int32)
    srcp = jnp.concatenate([src, (ar * 37) % N])
    dstp = jnp.concatenate([dst, N + (ar % 240)])
    dstk1 = dstp.reshape(32, 40, 128)
    dst3 = dstp.reshape(16, 80, 128)
    src4 = jnp.stack([srcp, srcp + N]).reshape(32, 80, 128)
    zeros640 = jnp.zeros((640,), jnp.float32)
    zeros128 = jnp.zeros((128, 128), jnp.float32)

    deg_kernel, msg_kernel = _sc_kernels()
    hist = deg_kernel(dstk1, zeros640).reshape(2, NPAD)[:, :N].T
    whs, norm = _k2(node_feats, W, b.reshape(2, 128), hist)
    h32 = msg_kernel(whs.reshape(2 * N, 128), src4, dst3, zeros128)
    return _k4(h32.reshape(2, N, 128), norm, node_feats)


# speedup vs baseline: 12.0365x; 1.0578x over previous
import functools

import jax
import jax.numpy as jnp
from jax import lax
from jax.experimental import pallas as pl
from jax.experimental.pallas import tpu as pltpu
from jax.experimental.pallas import tpu_sc as plsc

N = 10000
E = 160000
EP = 16 * 80 * 128
NPAD = 10240

def _deg_body(dstk, zeros640, out, dvm, ovm, vmf, vmc, dsh):
    c = lax.axis_index("c")
    s = lax.axis_index("s")
    wid = c * 16 + s
    ones16 = jnp.ones((16,), jnp.float32)

    @pl.loop(0, 8)
    def _(k):
        ovm[pl.ds(k * 16, 16)] = ones16

    pltpu.sync_copy(zeros640, dsh.at[pl.ds(s * 640, 640)])
    pltpu.sync_copy(dstk.at[wid], dvm)
    plsc.subcore_barrier()

    @pl.loop(0, 40)
    def _(j):
        pltpu.sync_copy(ovm, dsh.at[dvm.at[j]], add=True)

    plsc.subcore_barrier()
    pltpu.sync_copy(dsh.at[pl.ds(s * 640, 640)], vmf)

    @pl.loop(0, 40)
    def _(i):
        vmc[i // 8, pl.ds((i % 8) * 16, 16)] = vmf[pl.ds(i * 16, 16)]

    pltpu.sync_copy(vmc, out.at[wid])


def _k2_body(x_ref, w_ref, b_ref, hist_ref, whs_ref, norm_ref):
    c = pl.program_id(1)
    degs = jnp.sum(hist_ref[...], axis=0)
    norm = lax.rsqrt(jnp.maximum(degs, 1.0))
    acc = lax.dot_general(
        x_ref[...], w_ref[...], (((1,), (1,)), ((), ())),
        preferred_element_type=jnp.float32)
    whs_ref[0] = (acc + b_ref[pl.ds(c, 1), :]) * norm[:, None]
    norm_ref[...] = norm[:, None]


def _k2(x, w, b2, hist):
    return pl.pallas_call(
        _k2_body,
        grid=(5, 2),
        in_specs=[
            pl.BlockSpec((2048, 256), lambda i, c: (i, 0)),
            pl.BlockSpec((128, 256), lambda i, c: (c, 0)),
            pl.BlockSpec((2, 128), lambda i, c: (0, 0)),
            pl.BlockSpec((2, 2048), lambda i, c: (0, i)),
        ],
        out_specs=[
            pl.BlockSpec((1, 2048, 128), lambda i, c: (c, i, 0)),
            pl.BlockSpec((2048, 1), lambda i, c: (i, 0)),
        ],
        out_shape=[
            jax.ShapeDtypeStruct((2, NPAD, 128), jnp.float32),
            jax.ShapeDtypeStruct((N, 1), jnp.float32),
        ],
    )(x, w, b2, hist)


def _msg_body(whs_flat, src4, dst3, zeros128, out, svm, dvm, rows0, rows1,
              hsh, sg0, sg1, ss0, ss1):
    c = lax.axis_index("c")
    s = lax.axis_index("s")
    wid = c * 16 + s
    for k in range(5):
        pltpu.sync_copy(zeros128, hsh.at[pl.ds(s * 640 + k * 128, 128)])
    plsc.subcore_barrier()

    for p in range(2):
        pltpu.sync_copy(src4.at[wid, pl.ds(p * 40, 40)], svm)
        pltpu.sync_copy(dst3.at[s, pl.ds(p * 40, 40)], dvm)
        pltpu.async_copy(whs_flat.at[svm.at[0]], rows0, sg0)

        @pl.loop(0, 20)
        def _(j2):
            a = 2 * j2

            @pl.when(j2 > 0)
            def _():
                pltpu.make_async_copy(rows1, hsh.at[dvm.at[0]], ss1).wait()

            pltpu.async_copy(whs_flat.at[svm.at[a + 1]], rows1, sg1)
            pltpu.make_async_copy(whs_flat.at[svm.at[0]], rows0, sg0).wait()
            pltpu.async_copy(rows0, hsh.at[dvm.at[a]], ss0, add=True)

            @pl.when(j2 < 19)
            def _():
                pltpu.make_async_copy(rows0, hsh.at[dvm.at[0]], ss0).wait()
                pltpu.async_copy(whs_flat.at[svm.at[a + 2]], rows0, sg0)

            pltpu.make_async_copy(whs_flat.at[svm.at[0]], rows1, sg1).wait()
            pltpu.async_copy(rows1, hsh.at[dvm.at[a + 1]], ss1, add=True)

        pltpu.make_async_copy(rows0, hsh.at[dvm.at[0]], ss0).wait()
        pltpu.make_async_copy(rows1, hsh.at[dvm.at[0]], ss1).wait()

    plsc.subcore_barrier()
    pltpu.sync_copy(hsh.at[pl.ds(s * 625, 625)], out.at[wid])


def _k4_body(h2_ref, norm_ref, x_ref, o_ref):
    h = jnp.concatenate([h2_ref[0], h2_ref[1]], axis=1)
    o_ref[...] = jnp.maximum(h * norm_ref[...], 0.0) + x_ref[...]


@functools.lru_cache(maxsize=None)
def _sc_kernels():
    mesh = plsc.VectorSubcoreMesh(
        core_axis_name="c", subcore_axis_name="s", num_cores=2,
        num_subcores=16)
    deg = pl.kernel(
        _deg_body,
        mesh=mesh,
        out_type=jax.ShapeDtypeStruct((32, 5, 128), jnp.float32),
        scratch_types=[
            pltpu.VMEM((40, 128), jnp.int32),
            pltpu.VMEM((128,), jnp.float32),
            pltpu.VMEM((640,), jnp.float32),
            pltpu.VMEM((5, 128), jnp.float32),
            pltpu.VMEM_SHARED((NPAD,), jnp.float32),
        ],
    )
    msg = pl.kernel(
        _msg_body,
        mesh=mesh,
        out_type=jax.ShapeDtypeStruct((32, 625, 128), jnp.float32),
        scratch_types=[
            pltpu.VMEM((40, 128), jnp.int32),
            pltpu.VMEM((40, 128), jnp.int32),
            pltpu.VMEM((128, 128), jnp.float32),
            pltpu.VMEM((128, 128), jnp.float32),
            pltpu.VMEM_SHARED((NPAD, 128), jnp.float32),
            pltpu.SemaphoreType.DMA,
            pltpu.SemaphoreType.DMA,
            pltpu.SemaphoreType.DMA,
            pltpu.SemaphoreType.DMA,
        ],
    )
    return deg, msg


def _k4(h2, norm, x):
    return pl.pallas_call(
        _k4_body,
        grid=(5,),
        in_specs=[
            pl.BlockSpec((2, 2000, 128), lambda i: (0, i, 0)),
            pl.BlockSpec((2000, 1), lambda i: (i, 0)),
            pl.BlockSpec((2000, 256), lambda i: (i, 0)),
        ],
        out_specs=pl.BlockSpec((2000, 256), lambda i: (i, 0)),
        out_shape=jax.ShapeDtypeStruct((N, 256), jnp.float32),
    )(h2, norm, x)


def kernel(node_feats, edge_index, W, b):
    src = edge_index[0].astype(jnp.int32)
    dst = edge_index[1].astype(jnp.int32)
    pad = EP - E
    ar = jnp.arange(pad, dtype=jnp.int32)
    srcp = jnp.concatenate([src, (ar * 37) % N])
    dstp = jnp.concatenate([dst, N + (ar % 240)])
    dstk1 = dstp.reshape(32, 40, 128)
    dst3 = dstp.reshape(16, 80, 128)
    src4 = jnp.stack([srcp, srcp + NPAD]).reshape(32, 80, 128)
    zeros640 = jnp.zeros((640,), jnp.float32)
    zeros128 = jnp.zeros((128, 128), jnp.float32)

    deg_kernel, msg_kernel = _sc_kernels()
    hist = deg_kernel(dstk1, zeros640).reshape(2, NPAD)
    whs, norm = _k2(node_feats, W, b.reshape(2, 128), hist)
    h32 = msg_kernel(whs.reshape(2 * NPAD, 128), src4, dst3, zeros128)
    return _k4(h32.reshape(2, N, 128), norm, node_feats)


# speedup vs baseline: 12.5589x; 1.0434x over previous
import functools

import jax
import jax.numpy as jnp
from jax import lax
from jax.experimental import pallas as pl
from jax.experimental.pallas import tpu as pltpu
from jax.experimental.pallas import tpu_sc as plsc

N = 10000
E = 160000
EP = 16 * 80 * 128
NPAD = 10240

def _deg_body(dstk, zeros640, out, dvm, ovm, vmf, vmc, dsh):
    c = lax.axis_index("c")
    s = lax.axis_index("s")
    wid = c * 16 + s
    ones16 = jnp.ones((16,), jnp.float32)

    @pl.loop(0, 8)
    def _(k):
        ovm[pl.ds(k * 16, 16)] = ones16

    pltpu.sync_copy(zeros640, dsh.at[pl.ds(s * 640, 640)])
    pltpu.sync_copy(dstk.at[wid], dvm)
    plsc.subcore_barrier()

    @pl.loop(0, 40)
    def _(j):
        pltpu.sync_copy(ovm, dsh.at[dvm.at[j]], add=True)

    plsc.subcore_barrier()
    pltpu.sync_copy(dsh.at[pl.ds(s * 640, 640)], vmf)

    @pl.loop(0, 40)
    def _(i):
        vmc[i // 8, pl.ds((i % 8) * 16, 16)] = vmf[pl.ds(i * 16, 16)]

    pltpu.sync_copy(vmc, out.at[wid])


def _k2_body(x_ref, w_ref, b_ref, hist_ref, whs_ref, norm_ref):
    degs = jnp.sum(hist_ref[...], axis=0)
    norm = lax.rsqrt(jnp.maximum(degs, 1.0))
    acc = lax.dot_general(
        x_ref[...], w_ref[...], (((1,), (1,)), ((), ())),
        preferred_element_type=jnp.float32)
    wb = (acc + b_ref[...].reshape(1, 256)) * norm[:, None]
    whs_ref[0] = wb[:, :128]
    whs_ref[1] = wb[:, 128:]
    norm_ref[...] = norm[:, None]


def _k2(x, w, b2, hist):
    return pl.pallas_call(
        _k2_body,
        grid=(5,),
        in_specs=[
            pl.BlockSpec((2048, 256), lambda i: (i, 0)),
            pl.BlockSpec((256, 256), lambda i: (0, 0)),
            pl.BlockSpec((2, 128), lambda i: (0, 0)),
            pl.BlockSpec((2, 2048), lambda i: (0, i)),
        ],
        out_specs=[
            pl.BlockSpec((2, 2048, 128), lambda i: (0, i, 0)),
            pl.BlockSpec((2048, 1), lambda i: (i, 0)),
        ],
        out_shape=[
            jax.ShapeDtypeStruct((2, NPAD, 128), jnp.float32),
            jax.ShapeDtypeStruct((N, 1), jnp.float32),
        ],
    )(x, w, b2, hist)


def _msg_body(whs_flat, src4, dst3, zeros128, out, svm, dvm, rows0, rows1,
              hsh, sg0, sg1, ss0, ss1):
    c = lax.axis_index("c")
    s = lax.axis_index("s")
    wid = c * 16 + s
    for k in range(5):
        pltpu.async_copy(zeros128, hsh.at[pl.ds(s * 640 + k * 128, 128)], ss0)

    pltpu.sync_copy(src4.at[wid, pl.ds(0, 40)], svm)
    pltpu.sync_copy(dst3.at[s, pl.ds(0, 40)], dvm)
    pltpu.async_copy(whs_flat.at[svm.at[0]], rows0, sg0)
    for k in range(5):
        pltpu.make_async_copy(
            zeros128, hsh.at[pl.ds(s * 640 + k * 128, 128)], ss0).wait()
    plsc.subcore_barrier()

    for p in range(2):
        if p:
            pltpu.sync_copy(src4.at[wid, pl.ds(p * 40, 40)], svm)
            pltpu.sync_copy(dst3.at[s, pl.ds(p * 40, 40)], dvm)
            pltpu.async_copy(whs_flat.at[svm.at[0]], rows0, sg0)

        @pl.loop(0, 20)
        def _(j2):
            a = 2 * j2

            @pl.when(j2 > 0)
            def _():
                pltpu.make_async_copy(rows1, hsh.at[dvm.at[0]], ss1).wait()

            pltpu.async_copy(whs_flat.at[svm.at[a + 1]], rows1, sg1)
            pltpu.make_async_copy(whs_flat.at[svm.at[0]], rows0, sg0).wait()
            pltpu.async_copy(rows0, hsh.at[dvm.at[a]], ss0, add=True)

            @pl.when(j2 < 19)
            def _():
                pltpu.make_async_copy(rows0, hsh.at[dvm.at[0]], ss0).wait()
                pltpu.async_copy(whs_flat.at[svm.at[a + 2]], rows0, sg0)

            pltpu.make_async_copy(whs_flat.at[svm.at[0]], rows1, sg1).wait()
            pltpu.async_copy(rows1, hsh.at[dvm.at[a + 1]], ss1, add=True)

        pltpu.make_async_copy(rows0, hsh.at[dvm.at[0]], ss0).wait()
        pltpu.make_async_copy(rows1, hsh.at[dvm.at[0]], ss1).wait()

    plsc.subcore_barrier()
    pltpu.sync_copy(hsh.at[pl.ds(s * 625, 625)], out.at[wid])


def _k4_body(h2_ref, norm_ref, x_ref, o_ref):
    h = jnp.concatenate([h2_ref[0], h2_ref[1]], axis=1)
    o_ref[...] = jnp.maximum(h * norm_ref[...], 0.0) + x_ref[...]


@functools.lru_cache(maxsize=None)
def _sc_kernels():
    mesh = plsc.VectorSubcoreMesh(
        core_axis_name="c", subcore_axis_name="s", num_cores=2,
        num_subcores=16)
    deg = pl.kernel(
        _deg_body,
        mesh=mesh,
        out_type=jax.ShapeDtypeStruct((32, 5, 128), jnp.float32),
        scratch_types=[
            pltpu.VMEM((40, 128), jnp.int32),
            pltpu.VMEM((128,), jnp.float32),
            pltpu.VMEM((640,), jnp.float32),
            pltpu.VMEM((5, 128), jnp.float32),
            pltpu.VMEM_SHARED((NPAD,), jnp.float32),
        ],
    )
    msg = pl.kernel(
        _msg_body,
        mesh=mesh,
        out_type=jax.ShapeDtypeStruct((32, 625, 128), jnp.float32),
        scratch_types=[
            pltpu.VMEM((40, 128), jnp.int32),
            pltpu.VMEM((40, 128), jnp.int32),
            pltpu.VMEM((128, 128), jnp.float32),
            pltpu.VMEM((128, 128), jnp.float32),
            pltpu.VMEM_SHARED((NPAD, 128), jnp.float32),
            pltpu.SemaphoreType.DMA,
            pltpu.SemaphoreType.DMA,
            pltpu.SemaphoreType.DMA,
            pltpu.SemaphoreType.DMA,
        ],
    )
    return deg, msg


def _k4(h2, norm, x):
    return pl.pallas_call(
        _k4_body,
        grid=(5,),
        in_specs=[
            pl.BlockSpec((2, 2000, 128), lambda i: (0, i, 0)),
            pl.BlockSpec((2000, 1), lambda i: (i, 0)),
            pl.BlockSpec((2000, 256), lambda i: (i, 0)),
        ],
        out_specs=pl.BlockSpec((2000, 256), lambda i: (i, 0)),
        out_shape=jax.ShapeDtypeStruct((N, 256), jnp.float32),
    )(h2, norm, x)


def kernel(node_feats, edge_index, W, b):
    src = edge_index[0].astype(jnp.int32)
    dst = edge_index[1].astype(jnp.int32)
    pad = EP - E
    ar = jnp.arange(pad, dtype=jnp.int32)
    srcp = jnp.concatenate([src, (ar * 37) % N])
    dstp = jnp.concatenate([dst, N + (ar % 240)])
    dstk1 = dstp.reshape(32, 40, 128)
    dst3 = dstp.reshape(16, 80, 128)
    src4 = jnp.stack([srcp, srcp + NPAD]).reshape(32, 80, 128)
    zeros640 = jnp.zeros((640,), jnp.float32)
    zeros128 = jnp.zeros((128, 128), jnp.float32)

    deg_kernel, msg_kernel = _sc_kernels()
    hist = deg_kernel(dstk1, zeros640).reshape(2, NPAD)
    whs, norm = _k2(node_feats, W, b.reshape(2, 128), hist)
    h32 = msg_kernel(whs.reshape(2 * NPAD, 128), src4, dst3, zeros128)
    return _k4(h32.reshape(2, N, 128), norm, node_feats)


# speedup vs baseline: 12.9410x; 1.0304x over previous
import functools

import jax
import jax.numpy as jnp
from jax import lax
from jax.experimental import pallas as pl
from jax.experimental.pallas import tpu as pltpu
from jax.experimental.pallas import tpu_sc as plsc

N = 10000
E = 160000
EP = 16 * 80 * 128
NPAD = 10240

def _deg_body(dstk, zeros640, out, dvm, ovm, vmf, vmc, dsh, sa, sb):
    c = lax.axis_index("c")
    s = lax.axis_index("s")
    wid = c * 16 + s
    ones16 = jnp.ones((16,), jnp.float32)

    @pl.loop(0, 8)
    def _(k):
        ovm[pl.ds(k * 16, 16)] = ones16

    pltpu.sync_copy(zeros640, dsh.at[pl.ds(s * 640, 640)])
    pltpu.sync_copy(dstk.at[wid], dvm)
    plsc.subcore_barrier()

    @pl.loop(0, 20)
    def _(j2):
        a = 2 * j2

        @pl.when(j2 > 0)
        def _():
            pltpu.make_async_copy(ovm, dsh.at[dvm.at[0]], sa).wait()
            pltpu.make_async_copy(ovm, dsh.at[dvm.at[0]], sb).wait()

        pltpu.async_copy(ovm, dsh.at[dvm.at[a]], sa, add=True)
        pltpu.async_copy(ovm, dsh.at[dvm.at[a + 1]], sb, add=True)

    pltpu.make_async_copy(ovm, dsh.at[dvm.at[0]], sa).wait()
    pltpu.make_async_copy(ovm, dsh.at[dvm.at[0]], sb).wait()
    plsc.subcore_barrier()
    pltpu.sync_copy(dsh.at[pl.ds(s * 640, 640)], vmf)

    @pl.loop(0, 40)
    def _(i):
        vmc[i // 8, pl.ds((i % 8) * 16, 16)] = vmf[pl.ds(i * 16, 16)]

    pltpu.sync_copy(vmc, out.at[wid])


def _k2_body(x_ref, w_ref, b_ref, hist_ref, whs_ref, norm_ref):
    degs = jnp.sum(hist_ref[...], axis=0)
    norm = lax.rsqrt(jnp.maximum(degs, 1.0))
    acc = lax.dot_general(
        x_ref[...], w_ref[...], (((1,), (1,)), ((), ())),
        preferred_element_type=jnp.float32)
    wb = (acc + b_ref[...].reshape(1, 256)) * norm[:, None]
    whs_ref[0] = wb[:, :128]
    whs_ref[1] = wb[:, 128:]
    norm_ref[...] = norm[:, None]


def _k2(x, w, b2, hist):
    return pl.pallas_call(
        _k2_body,
        grid=(5,),
        in_specs=[
            pl.BlockSpec((2048, 256), lambda i: (i, 0)),
            pl.BlockSpec((256, 256), lambda i: (0, 0)),
            pl.BlockSpec((2, 128), lambda i: (0, 0)),
            pl.BlockSpec((2, 2048), lambda i: (0, i)),
        ],
        out_specs=[
            pl.BlockSpec((2, 2048, 128), lambda i: (0, i, 0)),
            pl.BlockSpec((2048, 1), lambda i: (i, 0)),
        ],
        out_shape=[
            jax.ShapeDtypeStruct((2, NPAD, 128), jnp.float32),
            jax.ShapeDtypeStruct((N, 1), jnp.float32),
        ],
    )(x, w, b2, hist)


def _msg_body(whs_flat, src4, dst3, zeros128, out, svm, dvm, rows0, rows1,
              hsh, sg0, sg1, ss0, ss1):
    c = lax.axis_index("c")
    s = lax.axis_index("s")
    wid = c * 16 + s
    for k in range(5):
        pltpu.async_copy(zeros128, hsh.at[pl.ds(s * 640 + k * 128, 128)], ss0)

    pltpu.sync_copy(src4.at[wid, pl.ds(0, 40)], svm)
    pltpu.sync_copy(dst3.at[s, pl.ds(0, 40)], dvm)
    pltpu.async_copy(whs_flat.at[svm.at[0]], rows0, sg0)
    for k in range(5):
        pltpu.make_async_copy(
            zeros128, hsh.at[pl.ds(s * 640 + k * 128, 128)], ss0).wait()
    plsc.subcore_barrier()

    for p in range(2):
        if p:
            pltpu.sync_copy(src4.at[wid, pl.ds(p * 40, 40)], svm)
            pltpu.sync_copy(dst3.at[s, pl.ds(p * 40, 40)], dvm)
            pltpu.async_copy(whs_flat.at[svm.at[0]], rows0, sg0)

        @pl.loop(0, 20)
        def _(j2):
            a = 2 * j2

            @pl.when(j2 > 0)
            def _():
                pltpu.make_async_copy(rows1, hsh.at[dvm.at[0]], ss1).wait()

            pltpu.async_copy(whs_flat.at[svm.at[a + 1]], rows1, sg1)
            pltpu.make_async_copy(whs_flat.at[svm.at[0]], rows0, sg0).wait()
            pltpu.async_copy(rows0, hsh.at[dvm.at[a]], ss0, add=True)

            @pl.when(j2 < 19)
            def _():
                pltpu.make_async_copy(rows0, hsh.at[dvm.at[0]], ss0).wait()
                pltpu.async_copy(whs_flat.at[svm.at[a + 2]], rows0, sg0)

            pltpu.make_async_copy(whs_flat.at[svm.at[0]], rows1, sg1).wait()
            pltpu.async_copy(rows1, hsh.at[dvm.at[a + 1]], ss1, add=True)

        pltpu.make_async_copy(rows0, hsh.at[dvm.at[0]], ss0).wait()
        pltpu.make_async_copy(rows1, hsh.at[dvm.at[0]], ss1).wait()

    plsc.subcore_barrier()
    pltpu.sync_copy(hsh.at[pl.ds(s * 625, 625)], out.at[wid])


def _k4_body(h2_ref, norm_ref, x_ref, o_ref):
    h = jnp.concatenate([h2_ref[0], h2_ref[1]], axis=1)
    o_ref[...] = jnp.maximum(h * norm_ref[...], 0.0) + x_ref[...]


@functools.lru_cache(maxsize=None)
def _sc_kernels():
    mesh = plsc.VectorSubcoreMesh(
        core_axis_name="c", subcore_axis_name="s", num_cores=2,
        num_subcores=16)
    deg = pl.kernel(
        _deg_body,
        mesh=mesh,
        out_type=jax.ShapeDtypeStruct((32, 5, 128), jnp.float32),
        scratch_types=[
            pltpu.VMEM((40, 128), jnp.int32),
            pltpu.VMEM((128,), jnp.float32),
            pltpu.VMEM((640,), jnp.float32),
            pltpu.VMEM((5, 128), jnp.float32),
            pltpu.VMEM_SHARED((NPAD,), jnp.float32),
            pltpu.SemaphoreType.DMA,
            pltpu.SemaphoreType.DMA,
        ],
    )
    msg = pl.kernel(
        _msg_body,
        mesh=mesh,
        out_type=jax.ShapeDtypeStruct((32, 625, 128), jnp.float32),
        scratch_types=[
            pltpu.VMEM((40, 128), jnp.int32),
            pltpu.VMEM((40, 128), jnp.int32),
            pltpu.VMEM((128, 128), jnp.float32),
            pltpu.VMEM((128, 128), jnp.float32),
            pltpu.VMEM_SHARED((NPAD, 128), jnp.float32),
            pltpu.SemaphoreType.DMA,
            pltpu.SemaphoreType.DMA,
            pltpu.SemaphoreType.DMA,
            pltpu.SemaphoreType.DMA,
        ],
    )
    return deg, msg


def _k4(h2, norm, x):
    return pl.pallas_call(
        _k4_body,
        grid=(5,),
        in_specs=[
            pl.BlockSpec((2, 2000, 128), lambda i: (0, i, 0)),
            pl.BlockSpec((2000, 1), lambda i: (i, 0)),
            pl.BlockSpec((2000, 256), lambda i: (i, 0)),
        ],
        out_specs=pl.BlockSpec((2000, 256), lambda i: (i, 0)),
        out_shape=jax.ShapeDtypeStruct((N, 256), jnp.float32),
    )(h2, norm, x)


def kernel(node_feats, edge_index, W, b):
    pad = EP - E
    ar = jnp.arange(pad, dtype=jnp.int32)
    pads = jnp.stack([(ar * 37) % N, N + (ar % 240)])
    eip = jnp.concatenate([edge_index.astype(jnp.int32), pads], axis=1)
    srcp, dstp = eip[0], eip[1]
    dstk1 = dstp.reshape(32, 40, 128)
    dst3 = dstp.reshape(16, 80, 128)
    src4 = (srcp[None, :] + jnp.array([[0], [NPAD]], jnp.int32)).reshape(
        32, 80, 128)
    zeros640 = jnp.zeros((640,), jnp.float32)
    zeros128 = jnp.zeros((128, 128), jnp.float32)

    deg_kernel, msg_kernel = _sc_kernels()
    hist = deg_kernel(dstk1, zeros640).reshape(2, NPAD)
    whs, norm = _k2(node_feats, W, b.reshape(2, 128), hist)
    h32 = msg_kernel(whs.reshape(2 * NPAD, 128), src4, dst3, zeros128)
    return _k4(h32.reshape(2, N, 128), norm, node_feats)
